# baseline (speedup 1.0000x reference)
import functools

import jax
import jax.numpy as jnp
from jax import lax
from jax.experimental import pallas as pl
from jax.experimental.pallas import tpu as pltpu

NG_FIXED = 45000
H = 64
DEPTH = 5
BE = 2048
BN = 512
EALIGN = 4096
NPAD = 50176


def _pad_rows(a, npad):
    return jnp.pad(a, ((0, npad - a.shape[0]),) + ((0, 0),) * (a.ndim - 1))


def _embed_body(x_ref, pos_ref, noise_ref, gm_ref, embW_ref, embb_ref,
                A_ref, B_ref, be1_ref,
                h_ref, posout_ref, hA_ref, hB_ref):
    h = x_ref[...] * embW_ref[...] + embb_ref[...]
    h_ref[...] = h
    posout_ref[...] = pos_ref[...] + (1.0 - gm_ref[...]) * noise_ref[...] * 0.01
    hA_ref[...] = jnp.dot(h, A_ref[...], preferred_element_type=jnp.float32)
    hB_ref[...] = jnp.dot(h, B_ref[...], preferred_element_type=jnp.float32) + be1_ref[...]


def _edge_body(gsrc_ref, gdst_ref, psrc_ref, pdst_ref,
               c_ref, We2_ref, be2_ref, Wx1_ref, bx1_ref, Wx2t_ref, bx2_ref,
               m_ref, relw_ref):
    rel = psrc_ref[...] - pdst_ref[...]
    d2 = jnp.sum(rel * rel, axis=-1, keepdims=True)
    m1 = jnp.maximum(gsrc_ref[...] + gdst_ref[...] + d2 * c_ref[...], 0.0)
    m = jnp.maximum(
        jnp.dot(m1, We2_ref[...], preferred_element_type=jnp.float32) + be2_ref[...], 0.0)
    t = jnp.maximum(
        jnp.dot(m, Wx1_ref[...], preferred_element_type=jnp.float32) + bx1_ref[...], 0.0)
    w = jnp.sum(t * Wx2t_ref[...], axis=-1, keepdims=True) + bx2_ref[...]
    m_ref[0] = m[:, :32]
    m_ref[1] = m[:, 32:]
    relw_ref[...] = rel * w


def _node_body(h_ref, agg_ref, dpos_ref, pos_ref, gm_ref,
               Wh1h_ref, Wh1a_ref, bh1_ref, Wh2_ref, bh2_ref, lng_ref, lnb_ref,
               A_ref, B_ref, be1_ref,
               hout_ref, posout_ref, hA_ref, hB_ref, *, sub_layer, has_next):
    h = h_ref[...]
    agg = jnp.concatenate([agg_ref[0], agg_ref[1]], axis=-1)
    z = jnp.maximum(
        jnp.dot(h, Wh1h_ref[...], preferred_element_type=jnp.float32)
        + jnp.dot(agg, Wh1a_ref[...], preferred_element_type=jnp.float32)
        + bh1_ref[...], 0.0)
    hu = jnp.dot(z, Wh2_ref[...], preferred_element_type=jnp.float32) + bh2_ref[...]
    mu = jnp.mean(hu, axis=-1, keepdims=True)
    var = jnp.mean((hu - mu) ** 2, axis=-1, keepdims=True)
    hu = (hu - mu) * lax.rsqrt(var + 1e-5) * lng_ref[...] + lnb_ref[...]
    hnew = h + hu
    hout_ref[...] = hnew
    if sub_layer:
        posout_ref[...] = pos_ref[...] + dpos_ref[...]
    else:
        posout_ref[...] = pos_ref[...] + (1.0 - gm_ref[...]) * dpos_ref[...]
    if has_next:
        hA_ref[...] = jnp.dot(hnew, A_ref[...], preferred_element_type=jnp.float32)
        hB_ref[...] = jnp.dot(hnew, B_ref[...], preferred_element_type=jnp.float32) + be1_ref[...]


def _pred_body(h_ref, W1_ref, b1_ref, W2t_ref, b2_ref, out_ref):
    t = jnp.maximum(
        jnp.dot(h_ref[...], W1_ref[...], preferred_element_type=jnp.float32) + b1_ref[...], 0.0)
    out_ref[...] = jnp.sum(t * W2t_ref[...], axis=-1, keepdims=True) + b2_ref[...]


def _full(shape):
    nd = len(shape)
    return pl.BlockSpec(shape, lambda i: (0,) * nd)


def _embed_call(x, pos, noise, gmf, p):
    npad = x.shape[0]
    grid = npad // BN
    embW = p['emb_W'].reshape(1, H)
    embb = p['emb_b'].reshape(1, H)
    A = p['We1_A']
    B = p['We1_B']
    be1 = p['be1'].reshape(1, H)
    return pl.pallas_call(
        _embed_body,
        grid=(grid,),
        in_specs=[
            pl.BlockSpec((BN, 1), lambda i: (i, 0)),
            pl.BlockSpec((BN, 2), lambda i: (i, 0)),
            pl.BlockSpec((BN, 2), lambda i: (i, 0)),
            pl.BlockSpec((BN, 1), lambda i: (i, 0)),
            _full((1, H)), _full((1, H)), _full((H, H)), _full((H, H)), _full((1, H)),
        ],
        out_specs=[
            pl.BlockSpec((BN, H), lambda i: (i, 0)),
            pl.BlockSpec((BN, 2), lambda i: (i, 0)),
            pl.BlockSpec((BN, H), lambda i: (i, 0)),
            pl.BlockSpec((BN, H), lambda i: (i, 0)),
        ],
        out_shape=[
            jax.ShapeDtypeStruct((npad, H), jnp.float32),
            jax.ShapeDtypeStruct((npad, 2), jnp.float32),
            jax.ShapeDtypeStruct((npad, H), jnp.float32),
            jax.ShapeDtypeStruct((npad, H), jnp.float32),
        ],
    )(x, pos, noise, gmf, embW, embb, A, B, be1)


def _edge_call(gsrc, gdst, psrc, pdst, p):
    epad = gsrc.shape[0]
    grid = epad // BE
    c = p['We1_c']
    be2 = p['be2'].reshape(1, H)
    bx1 = p['bx1'].reshape(1, H)
    Wx2t = p['Wx2'].reshape(1, H)
    bx2 = p['bx2'].reshape(1, 1)
    return pl.pallas_call(
        _edge_body,
        grid=(grid,),
        in_specs=[
            pl.BlockSpec((BE, H), lambda i: (i, 0)),
            pl.BlockSpec((BE, H), lambda i: (i, 0)),
            pl.BlockSpec((BE, 2), lambda i: (i, 0)),
            pl.BlockSpec((BE, 2), lambda i: (i, 0)),
            _full((1, H)), _full((H, H)), _full((1, H)),
            _full((H, H)), _full((1, H)), _full((1, H)), _full((1, 1)),
        ],
        out_specs=[
            pl.BlockSpec((2, BE, 32), lambda i: (0, i, 0)),
            pl.BlockSpec((BE, 2), lambda i: (i, 0)),
        ],
        out_shape=[
            jax.ShapeDtypeStruct((2, epad, 32), jnp.float32),
            jax.ShapeDtypeStruct((epad, 2), jnp.float32),
        ],
    )(gsrc, gdst, psrc, pdst, c, p['We2'], be2, p['Wx1'], bx1, Wx2t, bx2)


def _node_call(h, agg, dpos, pos, gmf, p, pnext, sub_layer):
    npad = h.shape[0]
    grid = npad // BN
    has_next = pnext is not None
    bh1 = p['bh1'].reshape(1, H)
    bh2 = p['bh2'].reshape(1, H)
    lng = p['ln_g'].reshape(1, H)
    lnb = p['ln_b'].reshape(1, H)
    if has_next:
        A, B, be1 = pnext['We1_A'], pnext['We1_B'], pnext['be1'].reshape(1, H)
    else:
        A = jnp.zeros((H, H), jnp.float32)
        B = jnp.zeros((H, H), jnp.float32)
        be1 = jnp.zeros((1, H), jnp.float32)
    out_specs = [
        pl.BlockSpec((BN, H), lambda i: (i, 0)),
        pl.BlockSpec((BN, 2), lambda i: (i, 0)),
    ]
    out_shape = [
        jax.ShapeDtypeStruct((npad, H), jnp.float32),
        jax.ShapeDtypeStruct((npad, 2), jnp.float32),
    ]
    if has_next:
        out_specs += [pl.BlockSpec((BN, H), lambda i: (i, 0)),
                      pl.BlockSpec((BN, H), lambda i: (i, 0))]
        out_shape += [jax.ShapeDtypeStruct((npad, H), jnp.float32),
                      jax.ShapeDtypeStruct((npad, H), jnp.float32)]
    body = functools.partial(_node_body, sub_layer=sub_layer, has_next=has_next)
    if not has_next:
        def body(h_ref, agg_ref, dpos_ref, pos_ref, gm_ref, Wh1h_ref, Wh1a_ref,
                 bh1_ref, Wh2_ref, bh2_ref, lng_ref, lnb_ref, A_ref, B_ref, be1_ref,
                 hout_ref, posout_ref):
            _node_body(h_ref, agg_ref, dpos_ref, pos_ref, gm_ref, Wh1h_ref,
                       Wh1a_ref, bh1_ref, Wh2_ref, bh2_ref, lng_ref, lnb_ref,
                       A_ref, B_ref, be1_ref, hout_ref, posout_ref, None, None,
                       sub_layer=sub_layer, has_next=False)
    return pl.pallas_call(
        body,
        grid=(grid,),
        in_specs=[
            pl.BlockSpec((BN, H), lambda i: (i, 0)),
            pl.BlockSpec((2, BN, 32), lambda i: (0, i, 0)),
            pl.BlockSpec((BN, 2), lambda i: (i, 0)),
            pl.BlockSpec((BN, 2), lambda i: (i, 0)),
            pl.BlockSpec((BN, 1), lambda i: (i, 0)),
            _full((H, H)), _full((H, H)), _full((1, H)), _full((H, H)),
            _full((1, H)), _full((1, H)), _full((1, H)),
            _full((H, H)), _full((H, H)), _full((1, H)),
        ],
        out_specs=out_specs,
        out_shape=out_shape,
    )(h, agg, dpos, pos, gmf, p['Wh1_h'], p['Wh1_a'], bh1, p['Wh2'], bh2,
      lng, lnb, A, B, be1)


def _pred_call(h_tail, p):
    n = h_tail.shape[0]
    b1 = p['pred_b1'].reshape(1, H)
    W2t = p['pred_W2'].reshape(1, H)
    b2 = p['pred_b2'].reshape(1, 1)
    return pl.pallas_call(
        _pred_body,
        grid=(n // 1000,),
        in_specs=[
            pl.BlockSpec((1000, H), lambda i: (i, 0)),
            _full((H, H)), _full((1, H)), _full((1, H)), _full((1, 1)),
        ],
        out_specs=pl.BlockSpec((1000, 1), lambda i: (i, 0)),
        out_shape=jax.ShapeDtypeStruct((n, 1), jnp.float32),
    )(h_tail, p['pred_W1'], b1, W2t, b2)


def _gather_call(hA, hB, pos, src, dst):
    gsrc = hA[src]
    gdst = hB[dst]
    psrc = pos[src]
    pdst = pos[dst]
    return gsrc, gdst, psrc, pdst


def _scatter_call(m_split, relw, dst, npad):
    agg0 = jax.ops.segment_sum(m_split[0], dst, num_segments=npad)
    agg1 = jax.ops.segment_sum(m_split[1], dst, num_segments=npad)
    dpos = jax.ops.segment_sum(relw, dst, num_segments=npad)
    return jnp.stack([agg0, agg1]), dpos


def _prep_layer_params(p):
    q = dict(p)
    q['We1_A'] = p['We1'][:H]
    q['We1_B'] = p['We1'][H:2 * H]
    q['We1_c'] = p['We1'][2 * H:2 * H + 1]
    q['Wh1_h'] = p['Wh1'][:H]
    q['Wh1_a'] = p['Wh1'][H:]
    return q


def _pad_edges(ei):
    e = ei.shape[1]
    epad = ((e + EALIGN - 1) // EALIGN) * EALIGN
    src = jnp.pad(ei[0], (0, epad - e), constant_values=50000)
    dst = jnp.pad(ei[1], (0, epad - e), constant_values=50000)
    return src.astype(jnp.int32), dst.astype(jnp.int32)


def kernel(x, pos, ground_node, edge_index, node_subnode_index,
           subgraph_edge_index, noise, params):
    n = x.shape[0]
    npad = NPAD
    gmf = _pad_rows(ground_node.astype(jnp.float32).reshape(-1, 1), npad)
    xp = _pad_rows(x, npad)
    posp = _pad_rows(pos, npad)
    noisep = _pad_rows(noise, npad)

    edges = [_pad_edges(edge_index), _pad_edges(node_subnode_index),
             _pad_edges(subgraph_edge_index)]

    lp = [[_prep_layer_params(params[k][i]) for k in ('ground', 'g2s', 'sub')]
          for i in range(DEPTH)]

    h, posc, hA, hB = _embed_call(
        xp, posp, noisep, gmf,
        {'emb_W': params['emb_W'], 'emb_b': params['emb_b'],
         'We1_A': lp[0][0]['We1_A'], 'We1_B': lp[0][0]['We1_B'],
         'be1': lp[0][0]['be1']})

    for i in range(DEPTH):
        for j in range(3):
            p = lp[i][j]
            src, dst = edges[j]
            gsrc, gdst, psrc, pdst = _gather_call(hA, hB, posc, src, dst)
            m_split, relw = _edge_call(gsrc, gdst, psrc, pdst, p)
            agg, dpos = _scatter_call(m_split, relw, dst, npad)
            if j == 2 and i == DEPTH - 1:
                pnext = None
            elif j == 2:
                pnext = lp[i + 1][0]
            else:
                pnext = lp[i][j + 1]
            res = _node_call(h, agg, dpos, posc, gmf, p, pnext, sub_layer=(j == 2))
            if pnext is None:
                h, posc = res
            else:
                h, posc, hA, hB = res

    h_tail = h[NG_FIXED:n]
    sp_pos = posc[NG_FIXED:n]
    sp_h = _pred_call(h_tail, params)
    return sp_pos, sp_h


# speedup vs baseline: 3.2732x; 3.2732x over previous
import functools

import jax
import jax.numpy as jnp
from jax import lax
from jax.experimental import pallas as pl
from jax.experimental.pallas import tpu as pltpu
from jax.experimental.pallas import tpu_sc as plsc

NG_FIXED = 45000
H = 64
DEPTH = 5
BE = 2048
BN = 128
EALIGN = 4096
NPAD = 50048
TW = 80
MW = 36
NWORK = 32


def _pad_rows(a, npad):
    return jnp.pad(a, ((0, npad - a.shape[0]),) + ((0, 0),) * (a.ndim - 1))


def _embed_body(x_ref, pos_ref, noise_ref, gm_ref, embW_ref, embb_ref,
                A_ref, B_ref, be1_ref,
                h_ref, posout_ref, ta_ref, tb_ref):
    h = x_ref[...] * embW_ref[...] + embb_ref[...]
    h_ref[...] = h
    posout = pos_ref[...] + (1.0 - gm_ref[...]) * noise_ref[...] * 0.01
    posout_ref[...] = posout
    pad = jnp.zeros((h.shape[0], TW - H - 2), jnp.float32)
    hA = jnp.dot(h, A_ref[...], preferred_element_type=jnp.float32)
    hB = jnp.dot(h, B_ref[...], preferred_element_type=jnp.float32) + be1_ref[...]
    ta_ref[...] = jnp.concatenate([hA, posout, pad], axis=-1)
    tb_ref[...] = jnp.concatenate([hB, posout, pad], axis=-1)


def _edge_body(gsrc_ref, gdst_ref,
               c_ref, We2_ref, be2_ref, Wx1_ref, bx1_ref, Wx2t_ref, bx2_ref,
               m_ref):
    gsrc = gsrc_ref[...]
    gdst = gdst_ref[...]
    rel = gsrc[:, H:H + 2] - gdst[:, H:H + 2]
    d2 = jnp.sum(rel * rel, axis=-1, keepdims=True)
    m1 = jnp.maximum(gsrc[:, :H] + gdst[:, :H] + d2 * c_ref[...], 0.0)
    m = jnp.maximum(
        jnp.dot(m1, We2_ref[...], preferred_element_type=jnp.float32) + be2_ref[...], 0.0)
    t = jnp.maximum(
        jnp.dot(m, Wx1_ref[...], preferred_element_type=jnp.float32) + bx1_ref[...], 0.0)
    w = jnp.sum(t * Wx2t_ref[...], axis=-1, keepdims=True) + bx2_ref[...]
    relw = rel * w
    z2 = jnp.zeros((m.shape[0], 2), jnp.float32)
    z4 = jnp.zeros((m.shape[0], 4), jnp.float32)
    m_ref[0] = jnp.concatenate([m[:, :32], relw, z2], axis=-1)
    m_ref[1] = jnp.concatenate([m[:, 32:], z4], axis=-1)


def _node_body(h_ref, agg_ref, pos_ref, gm_ref,
               Wh1h_ref, Wh1a_ref, bh1_ref, Wh2_ref, bh2_ref, lng_ref, lnb_ref,
               A_ref, B_ref, be1_ref,
               hout_ref, posout_ref, ta_ref, tb_ref, *, sub_layer, has_next):
    h = h_ref[...]
    agg = jnp.concatenate([agg_ref[0, :, :32], agg_ref[1, :, :32]], axis=-1)
    dpos = agg_ref[0, :, 32:34]
    z = jnp.maximum(
        jnp.dot(h, Wh1h_ref[...], preferred_element_type=jnp.float32)
        + jnp.dot(agg, Wh1a_ref[...], preferred_element_type=jnp.float32)
        + bh1_ref[...], 0.0)
    hu = jnp.dot(z, Wh2_ref[...], preferred_element_type=jnp.float32) + bh2_ref[...]
    mu = jnp.mean(hu, axis=-1, keepdims=True)
    var = jnp.mean((hu - mu) ** 2, axis=-1, keepdims=True)
    hu = (hu - mu) * lax.rsqrt(var + 1e-5) * lng_ref[...] + lnb_ref[...]
    hnew = h + hu
    hout_ref[...] = hnew
    if sub_layer:
        posout = pos_ref[...] + dpos
    else:
        posout = pos_ref[...] + (1.0 - gm_ref[...]) * dpos
    posout_ref[...] = posout
    if has_next:
        pad = jnp.zeros((h.shape[0], TW - H - 2), jnp.float32)
        hA = jnp.dot(hnew, A_ref[...], preferred_element_type=jnp.float32)
        hB = jnp.dot(hnew, B_ref[...], preferred_element_type=jnp.float32) + be1_ref[...]
        ta_ref[...] = jnp.concatenate([hA, posout, pad], axis=-1)
        tb_ref[...] = jnp.concatenate([hB, posout, pad], axis=-1)


def _pred_body(h_ref, W1_ref, b1_ref, W2t_ref, b2_ref, out_ref):
    t = jnp.maximum(
        jnp.dot(h_ref[...], W1_ref[...], preferred_element_type=jnp.float32) + b1_ref[...], 0.0)
    out_ref[...] = jnp.sum(t * W2t_ref[...], axis=-1, keepdims=True) + b2_ref[...]


def _full(shape):
    nd = len(shape)
    return pl.BlockSpec(shape, lambda i: (0,) * nd)


def _embed_call(x, pos, noise, gmf, p):
    npad = x.shape[0]
    grid = npad // BN
    embW = p['emb_W'].reshape(1, H)
    embb = p['emb_b'].reshape(1, H)
    return pl.pallas_call(
        _embed_body,
        grid=(grid,),
        in_specs=[
            pl.BlockSpec((BN, 1), lambda i: (i, 0)),
            pl.BlockSpec((BN, 2), lambda i: (i, 0)),
            pl.BlockSpec((BN, 2), lambda i: (i, 0)),
            pl.BlockSpec((BN, 1), lambda i: (i, 0)),
            _full((1, H)), _full((1, H)), _full((H, H)), _full((H, H)), _full((1, H)),
        ],
        out_specs=[
            pl.BlockSpec((BN, H), lambda i: (i, 0)),
            pl.BlockSpec((BN, 2), lambda i: (i, 0)),
            pl.BlockSpec((BN, TW), lambda i: (i, 0)),
            pl.BlockSpec((BN, TW), lambda i: (i, 0)),
        ],
        out_shape=[
            jax.ShapeDtypeStruct((npad, H), jnp.float32),
            jax.ShapeDtypeStruct((npad, 2), jnp.float32),
            jax.ShapeDtypeStruct((npad, TW), jnp.float32),
            jax.ShapeDtypeStruct((npad, TW), jnp.float32),
        ],
    )(x, pos, noise, gmf, embW, embb, p['We1_A'], p['We1_B'], p['be1'].reshape(1, H))


def _edge_call(gsrc, gdst, p):
    epad = gsrc.shape[0]
    grid = epad // BE
    return pl.pallas_call(
        _edge_body,
        grid=(grid,),
        in_specs=[
            pl.BlockSpec((BE, TW), lambda i: (i, 0)),
            pl.BlockSpec((BE, TW), lambda i: (i, 0)),
            _full((1, H)), _full((H, H)), _full((1, H)),
            _full((H, H)), _full((1, H)), _full((1, H)), _full((1, 1)),
        ],
        out_specs=pl.BlockSpec((2, BE, MW), lambda i: (0, i, 0)),
        out_shape=jax.ShapeDtypeStruct((2, epad, MW), jnp.float32),
    )(gsrc, gdst, p['We1_c'], p['We2'], p['be2'].reshape(1, H),
      p['Wx1'], p['bx1'].reshape(1, H), p['Wx2'].reshape(1, H),
      p['bx2'].reshape(1, 1))


def _node_call(h, agg, pos, gmf, p, pnext, sub_layer):
    npad = h.shape[0]
    grid = npad // BN
    has_next = pnext is not None
    if has_next:
        A, B, be1 = pnext['We1_A'], pnext['We1_B'], pnext['be1'].reshape(1, H)
    else:
        A = jnp.zeros((H, H), jnp.float32)
        B = jnp.zeros((H, H), jnp.float32)
        be1 = jnp.zeros((1, H), jnp.float32)
    out_specs = [
        pl.BlockSpec((BN, H), lambda i: (i, 0)),
        pl.BlockSpec((BN, 2), lambda i: (i, 0)),
    ]
    out_shape = [
        jax.ShapeDtypeStruct((npad, H), jnp.float32),
        jax.ShapeDtypeStruct((npad, 2), jnp.float32),
    ]
    if has_next:
        out_specs += [pl.BlockSpec((BN, TW), lambda i: (i, 0)),
                      pl.BlockSpec((BN, TW), lambda i: (i, 0))]
        out_shape += [jax.ShapeDtypeStruct((npad, TW), jnp.float32),
                      jax.ShapeDtypeStruct((npad, TW), jnp.float32)]
    if has_next:
        body = functools.partial(_node_body, sub_layer=sub_layer, has_next=True)
    else:
        def body(h_ref, agg_ref, pos_ref, gm_ref, Wh1h_ref, Wh1a_ref,
                 bh1_ref, Wh2_ref, bh2_ref, lng_ref, lnb_ref, A_ref, B_ref, be1_ref,
                 hout_ref, posout_ref):
            _node_body(h_ref, agg_ref, pos_ref, gm_ref, Wh1h_ref,
                       Wh1a_ref, bh1_ref, Wh2_ref, bh2_ref, lng_ref, lnb_ref,
                       A_ref, B_ref, be1_ref, hout_ref, posout_ref, None, None,
                       sub_layer=sub_layer, has_next=False)
    return pl.pallas_call(
        body,
        grid=(grid,),
        in_specs=[
            pl.BlockSpec((BN, H), lambda i: (i, 0)),
            pl.BlockSpec((2, BN, MW), lambda i: (0, i, 0)),
            pl.BlockSpec((BN, 2), lambda i: (i, 0)),
            pl.BlockSpec((BN, 1), lambda i: (i, 0)),
            _full((H, H)), _full((H, H)), _full((1, H)), _full((H, H)),
            _full((1, H)), _full((1, H)), _full((1, H)),
            _full((H, H)), _full((H, H)), _full((1, H)),
        ],
        out_specs=out_specs,
        out_shape=out_shape,
    )(h, agg, pos, gmf, p['Wh1_h'], p['Wh1_a'], p['bh1'].reshape(1, H),
      p['Wh2'], p['bh2'].reshape(1, H), p['ln_g'].reshape(1, H),
      p['ln_b'].reshape(1, H), A, B, be1)


def _pred_call(h_tail, p):
    n = h_tail.shape[0]
    return pl.pallas_call(
        _pred_body,
        grid=(n // 1000,),
        in_specs=[
            pl.BlockSpec((1000, H), lambda i: (i, 0)),
            _full((H, H)), _full((1, H)), _full((1, H)), _full((1, 1)),
        ],
        out_specs=pl.BlockSpec((1000, 1), lambda i: (i, 0)),
        out_shape=jax.ShapeDtypeStruct((n, 1), jnp.float32),
    )(h_tail, p['pred_W1'], p['pred_b1'].reshape(1, H),
      p['pred_W2'].reshape(1, H), p['pred_b2'].reshape(1, 1))


def _pick_chunk(rpt):
    for d in (4, 2, 1):
        if rpt % d == 0:
            return d
    return 1


@functools.lru_cache(maxsize=None)
def _make_gather(ep, npad):
    rows = ep // 128
    rpt = rows // NWORK
    ch = _pick_chunk(rpt)
    nsch = rpt // ch
    mesh = plsc.VectorSubcoreMesh(core_axis_name="c", subcore_axis_name="s",
                                  num_cores=2, num_subcores=16)

    def body(ta, tb, src2, dst2, gsrc, gdst, sidx, didx, bufa, bufb, sem):
        c = lax.axis_index("c")
        s = lax.axis_index("s")
        wid = s * 2 + c
        row0 = wid * rpt

        def loop(i, carry):
            row = row0 + i * ch
            ebase = row * 128
            pltpu.sync_copy(src2.at[pl.ds(row, ch)], sidx)
            pltpu.sync_copy(dst2.at[pl.ds(row, ch)], didx)
            cps = []
            for k in range(ch):
                cps.append(pltpu.async_copy(
                    ta.at[sidx.at[k]], bufa.at[pl.ds(k * 128, 128)], sem))
                cps.append(pltpu.async_copy(
                    tb.at[didx.at[k]], bufb.at[pl.ds(k * 128, 128)], sem))
            for cp in cps:
                cp.wait()
            pltpu.sync_copy(bufa, gsrc.at[pl.ds(ebase, ch * 128)])
            pltpu.sync_copy(bufb, gdst.at[pl.ds(ebase, ch * 128)])
            return carry

        lax.fori_loop(0, nsch, loop, 0)

    return pl.kernel(
        body, mesh=mesh,
        compiler_params=pltpu.CompilerParams(use_tc_tiling_on_sc=False),
        out_type=[jax.ShapeDtypeStruct((ep, TW), jnp.float32),
                  jax.ShapeDtypeStruct((ep, TW), jnp.float32)],
        scratch_types=[
            pltpu.VMEM((ch, 128), jnp.int32),
            pltpu.VMEM((ch, 128), jnp.int32),
            pltpu.VMEM((ch * 128, TW), jnp.float32),
            pltpu.VMEM((ch * 128, TW), jnp.float32),
            pltpu.SemaphoreType.DMA,
        ])


@functools.lru_cache(maxsize=None)
def _make_scatter(ep, npad):
    rows = ep // 128
    rpt = rows // 16
    ch = 2 if rpt % 2 == 0 else 1
    nsch = rpt // ch
    stripe = npad // 16
    mesh = plsc.VectorSubcoreMesh(core_axis_name="c", subcore_axis_name="s",
                                  num_cores=2, num_subcores=16)

    def body(m3, dst2, zeros_hbm, agg, acc, didx, mbuf, sem):
        c = lax.axis_index("c")
        s = lax.axis_index("s")
        pltpu.sync_copy(zeros_hbm.at[pl.ds(s * stripe, stripe)],
                        acc.at[pl.ds(s * stripe, stripe)])
        plsc.subcore_barrier()

        def loop(i, carry):
            row = s * rpt + i * ch
            ebase = row * 128
            pltpu.sync_copy(dst2.at[pl.ds(row, ch)], didx)
            pltpu.sync_copy(m3.at[c, pl.ds(ebase, ch * 128)], mbuf)
            for k in range(ch):
                pltpu.sync_copy(mbuf.at[pl.ds(k * 128, 128)],
                                acc.at[didx.at[k]], add=True)
            return carry

        lax.fori_loop(0, nsch, loop, 0)
        plsc.subcore_barrier()
        pltpu.sync_copy(acc.at[pl.ds(s * stripe, stripe)],
                        agg.at[c, pl.ds(s * stripe, stripe)])

    return pl.kernel(
        body, mesh=mesh,
        compiler_params=pltpu.CompilerParams(use_tc_tiling_on_sc=False),
        out_type=jax.ShapeDtypeStruct((2, npad, MW), jnp.float32),
        scratch_types=[
            pltpu.VMEM_SHARED((npad, MW), jnp.float32),
            pltpu.VMEM((ch, 128), jnp.int32),
            pltpu.VMEM((ch * 128, MW), jnp.float32),
            pltpu.SemaphoreType.DMA,
        ])


def _prep_layer_params(p):
    q = dict(p)
    q['We1_A'] = p['We1'][:H]
    q['We1_B'] = p['We1'][H:2 * H]
    q['We1_c'] = p['We1'][2 * H:2 * H + 1]
    q['Wh1_h'] = p['Wh1'][:H]
    q['Wh1_a'] = p['Wh1'][H:]
    return q


def _pad_edges(ei):
    e = ei.shape[1]
    epad = ((e + EALIGN - 1) // EALIGN) * EALIGN
    src = jnp.pad(ei[0], (0, epad - e), constant_values=50000).astype(jnp.int32)
    dst = jnp.pad(ei[1], (0, epad - e), constant_values=50000).astype(jnp.int32)
    return src.reshape(epad // 128, 128), dst.reshape(epad // 128, 128)


def kernel(x, pos, ground_node, edge_index, node_subnode_index,
           subgraph_edge_index, noise, params):
    n = x.shape[0]
    npad = NPAD
    gmf = _pad_rows(ground_node.astype(jnp.float32).reshape(-1, 1), npad)
    xp = _pad_rows(x, npad)
    posp = _pad_rows(pos, npad)
    noisep = _pad_rows(noise, npad)
    zeros_acc = jnp.zeros((npad, MW), jnp.float32)

    edges = [_pad_edges(edge_index), _pad_edges(node_subnode_index),
             _pad_edges(subgraph_edge_index)]

    lp = [[_prep_layer_params(params[k][i]) for k in ('ground', 'g2s', 'sub')]
          for i in range(DEPTH)]

    h, posc, ta, tb = _embed_call(
        xp, posp, noisep, gmf,
        {'emb_W': params['emb_W'], 'emb_b': params['emb_b'],
         'We1_A': lp[0][0]['We1_A'], 'We1_B': lp[0][0]['We1_B'],
         'be1': lp[0][0]['be1']})

    for i in range(DEPTH):
        for j in range(3):
            p = lp[i][j]
            src2, dst2 = edges[j]
            ep = src2.shape[0] * 128
            gsrc, gdst = _make_gather(ep, npad)(ta, tb, src2, dst2)
            m3 = _edge_call(gsrc, gdst, p)
            agg = _make_scatter(ep, npad)(m3, dst2, zeros_acc)
            if j == 2 and i == DEPTH - 1:
                pnext = None
            elif j == 2:
                pnext = lp[i + 1][0]
            else:
                pnext = lp[i][j + 1]
            res = _node_call(h, agg, posc, gmf, p, pnext, sub_layer=(j == 2))
            if pnext is None:
                h, posc = res
            else:
                h, posc, ta, tb = res

    h_tail = h[NG_FIXED:n]
    sp_pos = posc[NG_FIXED:n]
    sp_h = _pred_call(h_tail, params)
    return sp_pos, sp_h
